# baseline (speedup 1.0000x reference)
import functools

import jax
import jax.numpy as jnp
import numpy as np
from jax import lax
from jax.experimental import pallas as pl
from jax.experimental.pallas import tpu as pltpu
from jax.experimental.pallas import tpu_sc as plsc

_HEADS = 12


def _col(ref, e):
    m = ref[...]
    lane = lax.broadcasted_iota(jnp.int32, m.shape, 1)
    return jnp.sum(jnp.where(lane == e, m, 0.0), axis=1, keepdims=True)


def _dispatch_kernel(nexp, t_tile, s_slots, xt_ref, gw_ref, gb_ref,
                     probs_ref, dstg_ref, tb_ref, offt_ref, endt_ref,
                     rend_ref):
    n = xt_ref.shape[1]
    nb = 2 * nexp
    f32, bf16, i32 = jnp.float32, jnp.bfloat16, jnp.int32

    lt = jnp.dot(gw_ref[...], xt_ref[...], preferred_element_type=f32)
    lt = lt + gb_ref[...]
    eidx = lax.broadcasted_iota(i32, lt.shape, 0)
    m1 = jnp.max(lt, axis=0, keepdims=True)
    idx1 = jnp.min(jnp.where(lt >= m1, eidx, nexp), axis=0, keepdims=True)
    sel1 = eidx == idx1
    lt2 = jnp.where(sel1, -1e30, lt)
    m2 = jnp.max(lt2, axis=0, keepdims=True)
    idx2 = jnp.min(jnp.where(lt2 >= m2, eidx, nexp), axis=0, keepdims=True)
    z = jnp.exp(m2 - m1)
    p1 = 1.0 / (1.0 + z)
    p2 = z / (1.0 + z)
    probs_ref[...] = jnp.concatenate([p1, p2], axis=0).T

    b0 = 2 * idx1
    b1 = 2 * idx2 + 1
    riota = lax.broadcasted_iota(i32, (nb, n), 0)
    sel0b = riota == b0
    sel1b = riota == b1
    m16 = (sel0b | sel1b).astype(bf16)
    ri = lax.broadcasted_iota(i32, (n, n), 0)
    ci = lax.broadcasted_iota(i32, (n, n), 1)
    ltri = (ri <= ci).astype(bf16)
    cum = jnp.dot(m16, ltri, preferred_element_type=f32)
    counts = jnp.sum(m16.astype(f32), axis=1, keepdims=True)
    pc = jnp.floor((counts + (t_tile - 1)) * (1.0 / t_tile)) * t_tile
    bi = lax.broadcasted_iota(i32, (nb, nb), 0)
    bj = lax.broadcasted_iota(i32, (nb, nb), 1)
    slt = (bj < bi).astype(bf16)
    offs = jnp.dot(slt, pc.astype(bf16), preferred_element_type=f32)
    rank0 = jnp.sum(jnp.where(sel0b, cum - 1.0, 0.0), axis=0, keepdims=True)
    rank1 = jnp.sum(jnp.where(sel1b, cum - 1.0, 0.0), axis=0, keepdims=True)
    o0 = jnp.sum(jnp.where(sel0b, offs, 0.0), axis=0, keepdims=True)
    o1 = jnp.sum(jnp.where(sel1b, offs, 0.0), axis=0, keepdims=True)
    dst0 = o0 + rank0
    dst1 = o1 + rank1
    dstg_ref[...] = jnp.concatenate([dst0, dst1], axis=1).astype(i32)

    nt = s_slots // t_tile
    offt_f = offs * (1.0 / t_tile)
    ntb = pc * (1.0 / t_tile)
    ti = lax.broadcasted_iota(i32, (nb, nt), 1).astype(f32)
    tb = jnp.sum((offt_f <= ti).astype(f32), axis=0, keepdims=True) - 1.0
    used = jnp.sum(ntb, axis=0, keepdims=True)
    ti1 = lax.broadcasted_iota(i32, (1, nt), 1).astype(f32)
    tb = jnp.where(ti1 < used, tb, -1.0)
    tb_ref[...] = tb.astype(i32)
    offt_ref[...] = offt_f.T.astype(i32)
    endt_ref[...] = (offt_f + ntb).T.astype(i32)
    rend_ref[...] = (offs + counts).T.astype(i32)


def _gqkv_kernel(tb_ref, xs_ref, g_ref, b_ref, wt_ref, inb_ref,
                 q_ref, k_ref, v_ref):
    @pl.when(tb_ref[pl.program_id(0)] >= 0)
    def _():
        xs = xs_ref[...]
        mu = jnp.mean(xs, axis=-1, keepdims=True)
        va = jnp.mean((xs - mu) ** 2, axis=-1, keepdims=True)
        h = (xs - mu) * lax.rsqrt(va + 1e-5) * g_ref[0] + b_ref[0]
        acc = jnp.dot(h.astype(jnp.bfloat16), wt_ref[0],
                      preferred_element_type=jnp.float32)
        qkv = (acc + inb_ref[0]).astype(jnp.bfloat16)
        e = qkv.shape[1] // 3
        q_ref[...] = qkv[:, :e]
        k_ref[...] = qkv[:, e:2 * e]
        v_ref[...] = qkv[:, 2 * e:]


def _attn_block(head_dim, t_tile, b, t0, t1, rend, q2, k_ref, v_ref):
    f32, bf16 = jnp.float32, jnp.bfloat16
    tq, chw = q2.shape
    nh = chw // head_dim
    scale = 1.0 / float(np.sqrt(head_dim))
    kvt = t_tile
    rowi = lax.broadcasted_iota(jnp.int32, (kvt, 1), 0)
    coli = lax.broadcasted_iota(jnp.int32, (1, kvt), 1)
    niter = t1 - t0

    def body(i, car):
        base = t0 * t_tile + i * kvt
        kt = k_ref[pl.ds(base, kvt), :]
        vt = v_ref[pl.ds(base, kvt), :]
        vt = jnp.where(base + rowi < rend, vt, jnp.bfloat16(0.0))
        vc = base + coli < rend
        new = []
        for hh in range(nh):
            sl = slice(hh * head_dim, (hh + 1) * head_dim)
            m_o, l_o, a_o = car[hh]
            s = lax.dot_general(q2[:, sl], kt[:, sl], (((1,), (1,)), ((), ())),
                                preferred_element_type=f32) * scale
            s = jnp.where(vc, s, -1e30)
            m_n = jnp.maximum(m_o, jnp.max(s, axis=-1, keepdims=True))
            corr = jnp.exp(m_o - m_n)
            p = jnp.exp(s - m_n)
            l_n = l_o * corr + jnp.sum(p, axis=-1, keepdims=True)
            a_n = a_o * corr + jnp.dot(p.astype(bf16), vt[:, sl],
                                       preferred_element_type=f32)
            new.append((m_n, l_n, a_n))
        return tuple(new)

    init = tuple((jnp.full((tq, 1), -1e30, f32), jnp.zeros((tq, 1), f32),
                  jnp.zeros((tq, head_dim), f32)) for _ in range(nh))
    fin = lax.fori_loop(0, niter, body, init)
    outs = [a / jnp.where(l == 0.0, 1.0, l) for (_, l, a) in fin]
    return jnp.concatenate(outs, axis=-1).astype(bf16)


def _gblock_kernel(head_dim, t_tile, tb_ref, offt_ref, endt_ref, rend_ref,
                   q_ref, k_hbm, v_hbm, xs_ref, owt_ref, ob_ref, g2_ref,
                   b2ln_ref, w1t_ref, b1_ref, w2t_ref, b2_ref, ys_ref,
                   k_scr, v_scr, sem):
    t = pl.program_id(0)

    @pl.when(t == 0)
    def _load_kv():
        pltpu.async_copy(k_hbm, k_scr, sem).wait()
        pltpu.async_copy(v_hbm, v_scr, sem).wait()

    b = tb_ref[t]

    @pl.when(b >= 0)
    def _():
        ao = _attn_block(head_dim, t_tile, b, offt_ref[b], endt_ref[b],
                         rend_ref[b], q_ref[...], k_scr, v_scr)
        o = jnp.dot(ao, owt_ref[0], preferred_element_type=jnp.float32)
        o = o + ob_ref[0]
        x1 = xs_ref[...] + o
        mu = jnp.mean(x1, axis=-1, keepdims=True)
        va = jnp.mean((x1 - mu) ** 2, axis=-1, keepdims=True)
        h2 = (x1 - mu) * lax.rsqrt(va + 1e-5) * g2_ref[0] + b2ln_ref[0]
        tt = jnp.dot(h2.astype(jnp.bfloat16), w1t_ref[0],
                     preferred_element_type=jnp.float32)
        tt = (tt + b1_ref[0]).astype(jnp.bfloat16)
        tt = tt * jnp.bfloat16(0.5) * (jnp.bfloat16(1.0) + lax.erf(
            tt * jnp.bfloat16(1.0 / np.sqrt(2.0))))
        mlp = jnp.dot(tt, w2t_ref[0], preferred_element_type=jnp.float32)
        ys_ref[...] = x1 + mlp + b2_ref[0]


def _combine_kernel(y0_ref, y1_ref, p_ref, out_ref):
    out_ref[...] = (_col(p_ref, 0) * y0_ref[...]
                    + _col(p_ref, 1) * y1_ref[...])


def _sc_scatter_x(xf, dstg, s_slots):
    n, d = xf.shape
    nwork, rpw = dstg.shape
    info = plsc.get_sparse_core_info()
    nc = info.num_cores
    mesh = plsc.VectorSubcoreMesh(core_axis_name="c", subcore_axis_name="s")

    @functools.partial(
        pl.kernel, mesh=mesh,
        out_type=jax.ShapeDtypeStruct((s_slots, d), xf.dtype),
        scratch_types=[
            pltpu.VMEM((rpw,), jnp.int32),
            pltpu.VMEM((rpw, d), xf.dtype),
            pltpu.SemaphoreType.DMA,
        ],
    )
    def k(x_hbm, idx_hbm, out_hbm, idx_v, rows_v, sem):
        wid = lax.axis_index("s") * nc + lax.axis_index("c")
        pltpu.sync_copy(idx_hbm.at[wid], idx_v)
        src = lax.rem(wid * rpw, n)
        pltpu.sync_copy(x_hbm.at[pl.ds(src, rpw)], rows_v)
        pltpu.async_copy(rows_v, out_hbm.at[idx_v], sem).wait()

    return k(xf, dstg)


def _sc_gather_y(ys, dstg):
    _, d = ys.shape
    nwork, rpw = dstg.shape
    info = plsc.get_sparse_core_info()
    nc = info.num_cores
    mesh = plsc.VectorSubcoreMesh(core_axis_name="c", subcore_axis_name="s")

    @functools.partial(
        pl.kernel, mesh=mesh,
        out_type=jax.ShapeDtypeStruct((nwork * rpw, d), ys.dtype),
        scratch_types=[
            pltpu.VMEM((rpw,), jnp.int32),
            pltpu.VMEM((rpw, d), ys.dtype),
            pltpu.SemaphoreType.DMA,
        ],
    )
    def k(ys_hbm, idx_hbm, out_hbm, idx_v, rows_v, sem):
        wid = lax.axis_index("s") * nc + lax.axis_index("c")
        pltpu.sync_copy(idx_hbm.at[wid], idx_v)
        pltpu.async_copy(ys_hbm.at[idx_v], rows_v, sem).wait()
        pltpu.sync_copy(rows_v, out_hbm.at[pl.ds(wid * rpw, rpw)])

    return k(ys, dstg)


def kernel(x, gate_w, gate_b, ln1g, ln1b, inw, inb, outw, outb,
           ln2g, ln2b, w1, b1, w2, b2):
    bsz, n, emb = x.shape
    nexp, three_e, _ = inw.shape
    hid = w1.shape[1]
    heads = _HEADS
    hd = emb // heads
    nb = 2 * nexp
    f32, bf16, i32 = jnp.float32, jnp.bfloat16, jnp.int32

    tt = 256 if n % 256 == 0 else 8
    s_slots = ((2 * n + nb * (tt - 1) + tt - 1) // tt) * tt + tt
    nt = s_slots // tt

    xf = x.reshape(n, emb)
    xt = jnp.swapaxes(xf, 0, 1)
    inwt = jnp.swapaxes(inw, 1, 2).astype(bf16)
    outwt = jnp.swapaxes(outw, 1, 2).astype(bf16)
    w1t = jnp.swapaxes(w1, 1, 2).astype(bf16)
    w2t = jnp.swapaxes(w2, 1, 2).astype(bf16)
    ln1g3 = ln1g.reshape(nexp, 1, emb)
    ln1b3 = ln1b.reshape(nexp, 1, emb)
    inb3 = inb.reshape(nexp, 1, three_e)
    outb3 = outb.reshape(nexp, 1, emb)
    ln2g3 = ln2g.reshape(nexp, 1, emb)
    ln2b3 = ln2b.reshape(nexp, 1, emb)
    b13 = b1.reshape(nexp, 1, hid)
    b23 = b2.reshape(nexp, 1, emb)
    gb2 = gate_b.reshape(nexp, 1)

    probs2, dstg, tb2, offt2, endt2, rend2 = pl.pallas_call(
        functools.partial(_dispatch_kernel, nexp, tt, s_slots),
        out_shape=[
            jax.ShapeDtypeStruct((n, 2), f32),
            jax.ShapeDtypeStruct((1, 2 * n), i32),
            jax.ShapeDtypeStruct((1, nt), i32),
            jax.ShapeDtypeStruct((1, nb), i32),
            jax.ShapeDtypeStruct((1, nb), i32),
            jax.ShapeDtypeStruct((1, nb), i32),
        ],
    )(xt, gate_w, gb2)
    tb = tb2.reshape(nt)
    offt = offt2.reshape(nb)
    endt = endt2.reshape(nb)
    rend = rend2.reshape(nb)

    nwork = 32
    dstg2 = dstg.reshape(nwork, (2 * n) // nwork)
    xs = _sc_scatter_x(xf, dstg2, s_slots)

    qkvs = pl.pallas_call(
        _gqkv_kernel,
        grid_spec=pltpu.PrefetchScalarGridSpec(
            num_scalar_prefetch=1,
            grid=(nt,),
            in_specs=[
                pl.BlockSpec((tt, emb), lambda t, tb: (t, 0)),
                pl.BlockSpec((1, 1, emb), lambda t, tb: (jnp.maximum(tb[t], 0) // 2, 0, 0)),
                pl.BlockSpec((1, 1, emb), lambda t, tb: (jnp.maximum(tb[t], 0) // 2, 0, 0)),
                pl.BlockSpec((1, emb, three_e), lambda t, tb: (jnp.maximum(tb[t], 0) // 2, 0, 0)),
                pl.BlockSpec((1, 1, three_e), lambda t, tb: (jnp.maximum(tb[t], 0) // 2, 0, 0)),
            ],
            out_specs=[
                pl.BlockSpec((tt, emb), lambda t, tb: (t, 0)),
                pl.BlockSpec((tt, emb), lambda t, tb: (t, 0)),
                pl.BlockSpec((tt, emb), lambda t, tb: (t, 0)),
            ],
        ),
        out_shape=[
            jax.ShapeDtypeStruct((s_slots, emb), bf16),
            jax.ShapeDtypeStruct((s_slots, emb), bf16),
            jax.ShapeDtypeStruct((s_slots, emb), bf16),
        ],
    )(tb, xs, ln1g3, ln1b3, inwt, inb3)
    qs, ks, vs = qkvs

    ys = pl.pallas_call(
        functools.partial(_gblock_kernel, hd, tt),
        grid_spec=pltpu.PrefetchScalarGridSpec(
            num_scalar_prefetch=4,
            grid=(nt,),
            in_specs=[
                pl.BlockSpec((tt, emb), lambda t, *_: (t, 0)),
                pl.BlockSpec(memory_space=pl.ANY),
                pl.BlockSpec(memory_space=pl.ANY),
                pl.BlockSpec((tt, emb), lambda t, *_: (t, 0)),
                pl.BlockSpec((1, emb, emb), lambda t, tb, *_: (jnp.maximum(tb[t], 0) // 2, 0, 0)),
                pl.BlockSpec((1, 1, emb), lambda t, tb, *_: (jnp.maximum(tb[t], 0) // 2, 0, 0)),
                pl.BlockSpec((1, 1, emb), lambda t, tb, *_: (jnp.maximum(tb[t], 0) // 2, 0, 0)),
                pl.BlockSpec((1, 1, emb), lambda t, tb, *_: (jnp.maximum(tb[t], 0) // 2, 0, 0)),
                pl.BlockSpec((1, emb, hid), lambda t, tb, *_: (jnp.maximum(tb[t], 0) // 2, 0, 0)),
                pl.BlockSpec((1, 1, hid), lambda t, tb, *_: (jnp.maximum(tb[t], 0) // 2, 0, 0)),
                pl.BlockSpec((1, hid, emb), lambda t, tb, *_: (jnp.maximum(tb[t], 0) // 2, 0, 0)),
                pl.BlockSpec((1, 1, emb), lambda t, tb, *_: (jnp.maximum(tb[t], 0) // 2, 0, 0)),
            ],
            out_specs=pl.BlockSpec((tt, emb), lambda t, *_: (t, 0)),
            scratch_shapes=[
                pltpu.VMEM((s_slots, emb), bf16),
                pltpu.VMEM((s_slots, emb), bf16),
                pltpu.SemaphoreType.DMA,
            ],
        ),
        out_shape=jax.ShapeDtypeStruct((s_slots, emb), f32),
        compiler_params=pltpu.CompilerParams(
            vmem_limit_bytes=64 * 1024 * 1024),
    )(tb, offt, endt, rend, qs, ks, vs, xs, outwt, outb3, ln2g3, ln2b3,
      w1t, b13, w2t, b23)

    yg = _sc_gather_y(ys, dstg2)

    rt = min(1024, n)
    nrt = n // rt
    out = pl.pallas_call(
        _combine_kernel,
        grid=(nrt,),
        in_specs=[
            pl.BlockSpec((rt, emb), lambda t: (t, 0)),
            pl.BlockSpec((rt, emb), lambda t: (t + nrt, 0)),
            pl.BlockSpec((rt, 2), lambda t: (t, 0)),
        ],
        out_specs=pl.BlockSpec((rt, emb), lambda t: (t, 0)),
        out_shape=jax.ShapeDtypeStruct((n, emb), f32),
    )(yg, yg, probs2)

    return out.reshape(bsz, n, emb)


# speedup vs baseline: 1.0164x; 1.0164x over previous
import functools

import jax
import jax.numpy as jnp
import numpy as np
from jax import lax
from jax.experimental import pallas as pl
from jax.experimental.pallas import tpu as pltpu
from jax.experimental.pallas import tpu_sc as plsc

_HEADS = 12


def _col(ref, e):
    m = ref[...]
    lane = lax.broadcasted_iota(jnp.int32, m.shape, 1)
    return jnp.sum(jnp.where(lane == e, m, 0.0), axis=1, keepdims=True)


def _dispatch_kernel(nexp, t_tile, s_slots, xt_ref, gw_ref, gb_ref,
                     probs_ref, dstg_ref, tb_ref, offt_ref, endt_ref,
                     rend_ref):
    n = xt_ref.shape[1]
    nb = 2 * nexp
    f32, bf16, i32 = jnp.float32, jnp.bfloat16, jnp.int32

    lt = jnp.dot(gw_ref[...], xt_ref[...], preferred_element_type=f32)
    lt = lt + gb_ref[...]
    eidx = lax.broadcasted_iota(i32, lt.shape, 0)
    m1 = jnp.max(lt, axis=0, keepdims=True)
    idx1 = jnp.min(jnp.where(lt >= m1, eidx, nexp), axis=0, keepdims=True)
    sel1 = eidx == idx1
    lt2 = jnp.where(sel1, -1e30, lt)
    m2 = jnp.max(lt2, axis=0, keepdims=True)
    idx2 = jnp.min(jnp.where(lt2 >= m2, eidx, nexp), axis=0, keepdims=True)
    z = jnp.exp(m2 - m1)
    p1 = 1.0 / (1.0 + z)
    p2 = z / (1.0 + z)
    probs_ref[...] = jnp.concatenate([p1, p2], axis=0).T

    b0 = 2 * idx1
    b1 = 2 * idx2 + 1
    riota = lax.broadcasted_iota(i32, (nb, n), 0)
    sel0b = riota == b0
    sel1b = riota == b1
    m16 = (sel0b | sel1b).astype(bf16)
    ri = lax.broadcasted_iota(i32, (n, n), 0)
    ci = lax.broadcasted_iota(i32, (n, n), 1)
    ltri = (ri <= ci).astype(bf16)
    cum = jnp.dot(m16, ltri, preferred_element_type=f32)
    counts = jnp.sum(m16.astype(f32), axis=1, keepdims=True)
    pc = jnp.floor((counts + (t_tile - 1)) * (1.0 / t_tile)) * t_tile
    bi = lax.broadcasted_iota(i32, (nb, nb), 0)
    bj = lax.broadcasted_iota(i32, (nb, nb), 1)
    slt = (bj < bi).astype(bf16)
    offs = jnp.dot(slt, pc.astype(bf16), preferred_element_type=f32)
    rank0 = jnp.sum(jnp.where(sel0b, cum - 1.0, 0.0), axis=0, keepdims=True)
    rank1 = jnp.sum(jnp.where(sel1b, cum - 1.0, 0.0), axis=0, keepdims=True)
    o0 = jnp.sum(jnp.where(sel0b, offs, 0.0), axis=0, keepdims=True)
    o1 = jnp.sum(jnp.where(sel1b, offs, 0.0), axis=0, keepdims=True)
    dst0 = o0 + rank0
    dst1 = o1 + rank1
    dstg_ref[...] = jnp.concatenate([dst0, dst1], axis=1).astype(i32)

    nt = s_slots // t_tile
    offt_f = offs * (1.0 / t_tile)
    ntb = pc * (1.0 / t_tile)
    ti = lax.broadcasted_iota(i32, (nb, nt), 1).astype(f32)
    tb = jnp.sum((offt_f <= ti).astype(f32), axis=0, keepdims=True) - 1.0
    used = jnp.sum(ntb, axis=0, keepdims=True)
    ti1 = lax.broadcasted_iota(i32, (1, nt), 1).astype(f32)
    tb = jnp.where(ti1 < used, tb, -1.0)
    tb_ref[...] = tb.astype(i32)
    offt_ref[...] = offt_f.T.astype(i32)
    endt_ref[...] = (offt_f + ntb).T.astype(i32)
    rend_ref[...] = (offs + counts).T.astype(i32)


def _gqkv_kernel(tb_ref, xs_ref, g_ref, b_ref, wt_ref, inb_ref,
                 q_ref, k_ref, v_ref):
    @pl.when(tb_ref[pl.program_id(0)] >= 0)
    def _():
        xs = xs_ref[...]
        mu = jnp.mean(xs, axis=-1, keepdims=True)
        va = jnp.mean((xs - mu) ** 2, axis=-1, keepdims=True)
        h = (xs - mu) * lax.rsqrt(va + 1e-5) * g_ref[0] + b_ref[0]
        acc = jnp.dot(h.astype(jnp.bfloat16), wt_ref[0],
                      preferred_element_type=jnp.float32)
        qkv = (acc + inb_ref[0]).astype(jnp.bfloat16)
        e = qkv.shape[1] // 3
        q_ref[...] = qkv[:, :e]
        k_ref[...] = qkv[:, e:2 * e]
        v_ref[...] = qkv[:, 2 * e:]


def _attn_block(head_dim, t_tile, b, t0, t1, rend, q2, k_ref, v_ref):
    f32, bf16 = jnp.float32, jnp.bfloat16
    tq, chw = q2.shape
    nh = chw // head_dim
    scale = 1.0 / float(np.sqrt(head_dim))
    kvt = 2 * t_tile
    rowi = lax.broadcasted_iota(jnp.int32, (kvt, 1), 0)
    coli = lax.broadcasted_iota(jnp.int32, (1, kvt), 1)
    niter = (t1 - t0 + 1) // 2

    def body(i, car):
        base = t0 * t_tile + i * kvt
        kt = k_ref[pl.ds(base, kvt), :]
        vt = v_ref[pl.ds(base, kvt), :]
        vt = jnp.where(base + rowi < rend, vt, jnp.bfloat16(0.0))
        vc = base + coli < rend
        new = []
        for hh in range(nh):
            sl = slice(hh * head_dim, (hh + 1) * head_dim)
            m_o, l_o, a_o = car[hh]
            s = lax.dot_general(q2[:, sl], kt[:, sl], (((1,), (1,)), ((), ())),
                                preferred_element_type=f32) * scale
            s = jnp.where(vc, s, -1e30)
            m_n = jnp.maximum(m_o, jnp.max(s, axis=-1, keepdims=True))
            corr = jnp.exp(m_o - m_n)
            p = jnp.exp(s - m_n)
            l_n = l_o * corr + jnp.sum(p, axis=-1, keepdims=True)
            a_n = a_o * corr + jnp.dot(p.astype(bf16), vt[:, sl],
                                       preferred_element_type=f32)
            new.append((m_n, l_n, a_n))
        return tuple(new)

    init = tuple((jnp.full((tq, 1), -1e30, f32), jnp.zeros((tq, 1), f32),
                  jnp.zeros((tq, head_dim), f32)) for _ in range(nh))
    fin = lax.fori_loop(0, niter, body, init)
    outs = [a / jnp.where(l == 0.0, 1.0, l) for (_, l, a) in fin]
    return jnp.concatenate(outs, axis=-1).astype(bf16)


def _gblock_kernel(head_dim, t_tile, tb_ref, offt_ref, endt_ref, rend_ref,
                   q_ref, k_hbm, v_hbm, xs_ref, owt_ref, ob_ref, g2_ref,
                   b2ln_ref, w1t_ref, b1_ref, w2t_ref, b2_ref, ys_ref,
                   k_scr, v_scr, sem):
    t = pl.program_id(0)

    @pl.when(t == 0)
    def _load_kv():
        pltpu.async_copy(k_hbm, k_scr, sem).wait()
        pltpu.async_copy(v_hbm, v_scr, sem).wait()

    b = tb_ref[t]

    @pl.when(b >= 0)
    def _():
        ao = _attn_block(head_dim, t_tile, b, offt_ref[b], endt_ref[b],
                         rend_ref[b], q_ref[...], k_scr, v_scr)
        o = jnp.dot(ao, owt_ref[0], preferred_element_type=jnp.float32)
        o = o + ob_ref[0]
        x1 = xs_ref[...] + o
        mu = jnp.mean(x1, axis=-1, keepdims=True)
        va = jnp.mean((x1 - mu) ** 2, axis=-1, keepdims=True)
        h2 = (x1 - mu) * lax.rsqrt(va + 1e-5) * g2_ref[0] + b2ln_ref[0]
        tt = jnp.dot(h2.astype(jnp.bfloat16), w1t_ref[0],
                     preferred_element_type=jnp.float32)
        tt = (tt + b1_ref[0]).astype(jnp.bfloat16)
        tt = tt * jnp.bfloat16(0.5) * (jnp.bfloat16(1.0) + lax.erf(
            tt * jnp.bfloat16(1.0 / np.sqrt(2.0))))
        mlp = jnp.dot(tt, w2t_ref[0], preferred_element_type=jnp.float32)
        ys_ref[...] = x1 + mlp + b2_ref[0]


def _combine_kernel(y0_ref, y1_ref, p_ref, out_ref):
    out_ref[...] = (_col(p_ref, 0) * y0_ref[...]
                    + _col(p_ref, 1) * y1_ref[...])


def _sc_scatter_x(xf, dstg, s_slots):
    n, d = xf.shape
    nwork, rpw = dstg.shape
    info = plsc.get_sparse_core_info()
    nc = info.num_cores
    mesh = plsc.VectorSubcoreMesh(core_axis_name="c", subcore_axis_name="s")

    @functools.partial(
        pl.kernel, mesh=mesh,
        out_type=jax.ShapeDtypeStruct((s_slots, d), xf.dtype),
        scratch_types=[
            pltpu.VMEM((rpw,), jnp.int32),
            pltpu.VMEM((rpw, d), xf.dtype),
            pltpu.SemaphoreType.DMA,
        ],
    )
    def k(x_hbm, idx_hbm, out_hbm, idx_v, rows_v, sem):
        wid = lax.axis_index("s") * nc + lax.axis_index("c")
        pltpu.sync_copy(idx_hbm.at[wid], idx_v)
        src = lax.rem(wid * rpw, n)
        pltpu.sync_copy(x_hbm.at[pl.ds(src, rpw)], rows_v)
        pltpu.async_copy(rows_v, out_hbm.at[idx_v], sem).wait()

    return k(xf, dstg)


def _sc_gather_y(ys, dstg):
    _, d = ys.shape
    nwork, rpw = dstg.shape
    info = plsc.get_sparse_core_info()
    nc = info.num_cores
    mesh = plsc.VectorSubcoreMesh(core_axis_name="c", subcore_axis_name="s")

    @functools.partial(
        pl.kernel, mesh=mesh,
        out_type=jax.ShapeDtypeStruct((nwork * rpw, d), ys.dtype),
        scratch_types=[
            pltpu.VMEM((rpw,), jnp.int32),
            pltpu.VMEM((rpw, d), ys.dtype),
            pltpu.SemaphoreType.DMA,
        ],
    )
    def k(ys_hbm, idx_hbm, out_hbm, idx_v, rows_v, sem):
        wid = lax.axis_index("s") * nc + lax.axis_index("c")
        pltpu.sync_copy(idx_hbm.at[wid], idx_v)
        pltpu.async_copy(ys_hbm.at[idx_v], rows_v, sem).wait()
        pltpu.sync_copy(rows_v, out_hbm.at[pl.ds(wid * rpw, rpw)])

    return k(ys, dstg)


def kernel(x, gate_w, gate_b, ln1g, ln1b, inw, inb, outw, outb,
           ln2g, ln2b, w1, b1, w2, b2):
    bsz, n, emb = x.shape
    nexp, three_e, _ = inw.shape
    hid = w1.shape[1]
    heads = _HEADS
    hd = emb // heads
    nb = 2 * nexp
    f32, bf16, i32 = jnp.float32, jnp.bfloat16, jnp.int32

    tt = 256 if n % 256 == 0 else 8
    s_slots = ((2 * n + nb * (tt - 1) + tt - 1) // tt) * tt + tt
    nt = s_slots // tt

    xf = x.reshape(n, emb)
    xt = jnp.swapaxes(xf, 0, 1)
    inwt = jnp.swapaxes(inw, 1, 2).astype(bf16)
    outwt = jnp.swapaxes(outw, 1, 2).astype(bf16)
    w1t = jnp.swapaxes(w1, 1, 2).astype(bf16)
    w2t = jnp.swapaxes(w2, 1, 2).astype(bf16)
    ln1g3 = ln1g.reshape(nexp, 1, emb)
    ln1b3 = ln1b.reshape(nexp, 1, emb)
    inb3 = inb.reshape(nexp, 1, three_e)
    outb3 = outb.reshape(nexp, 1, emb)
    ln2g3 = ln2g.reshape(nexp, 1, emb)
    ln2b3 = ln2b.reshape(nexp, 1, emb)
    b13 = b1.reshape(nexp, 1, hid)
    b23 = b2.reshape(nexp, 1, emb)
    gb2 = gate_b.reshape(nexp, 1)

    probs2, dstg, tb2, offt2, endt2, rend2 = pl.pallas_call(
        functools.partial(_dispatch_kernel, nexp, tt, s_slots),
        out_shape=[
            jax.ShapeDtypeStruct((n, 2), f32),
            jax.ShapeDtypeStruct((1, 2 * n), i32),
            jax.ShapeDtypeStruct((1, nt), i32),
            jax.ShapeDtypeStruct((1, nb), i32),
            jax.ShapeDtypeStruct((1, nb), i32),
            jax.ShapeDtypeStruct((1, nb), i32),
        ],
    )(xt, gate_w, gb2)
    tb = tb2.reshape(nt)
    offt = offt2.reshape(nb)
    endt = endt2.reshape(nb)
    rend = rend2.reshape(nb)

    nwork = 32
    dstg2 = dstg.reshape(nwork, (2 * n) // nwork)
    xs = _sc_scatter_x(xf, dstg2, s_slots)

    qkvs = pl.pallas_call(
        _gqkv_kernel,
        grid_spec=pltpu.PrefetchScalarGridSpec(
            num_scalar_prefetch=1,
            grid=(nt,),
            in_specs=[
                pl.BlockSpec((tt, emb), lambda t, tb: (t, 0)),
                pl.BlockSpec((1, 1, emb), lambda t, tb: (jnp.maximum(tb[t], 0) // 2, 0, 0)),
                pl.BlockSpec((1, 1, emb), lambda t, tb: (jnp.maximum(tb[t], 0) // 2, 0, 0)),
                pl.BlockSpec((1, emb, three_e), lambda t, tb: (jnp.maximum(tb[t], 0) // 2, 0, 0)),
                pl.BlockSpec((1, 1, three_e), lambda t, tb: (jnp.maximum(tb[t], 0) // 2, 0, 0)),
            ],
            out_specs=[
                pl.BlockSpec((tt, emb), lambda t, tb: (t, 0)),
                pl.BlockSpec((tt, emb), lambda t, tb: (t, 0)),
                pl.BlockSpec((tt, emb), lambda t, tb: (t, 0)),
            ],
        ),
        out_shape=[
            jax.ShapeDtypeStruct((s_slots, emb), bf16),
            jax.ShapeDtypeStruct((s_slots, emb), bf16),
            jax.ShapeDtypeStruct((s_slots, emb), bf16),
        ],
    )(tb, xs, ln1g3, ln1b3, inwt, inb3)
    qs, ks, vs = qkvs

    ys = pl.pallas_call(
        functools.partial(_gblock_kernel, hd, tt),
        grid_spec=pltpu.PrefetchScalarGridSpec(
            num_scalar_prefetch=4,
            grid=(nt,),
            in_specs=[
                pl.BlockSpec((tt, emb), lambda t, *_: (t, 0)),
                pl.BlockSpec(memory_space=pl.ANY),
                pl.BlockSpec(memory_space=pl.ANY),
                pl.BlockSpec((tt, emb), lambda t, *_: (t, 0)),
                pl.BlockSpec((1, emb, emb), lambda t, tb, *_: (jnp.maximum(tb[t], 0) // 2, 0, 0)),
                pl.BlockSpec((1, 1, emb), lambda t, tb, *_: (jnp.maximum(tb[t], 0) // 2, 0, 0)),
                pl.BlockSpec((1, 1, emb), lambda t, tb, *_: (jnp.maximum(tb[t], 0) // 2, 0, 0)),
                pl.BlockSpec((1, 1, emb), lambda t, tb, *_: (jnp.maximum(tb[t], 0) // 2, 0, 0)),
                pl.BlockSpec((1, emb, hid), lambda t, tb, *_: (jnp.maximum(tb[t], 0) // 2, 0, 0)),
                pl.BlockSpec((1, 1, hid), lambda t, tb, *_: (jnp.maximum(tb[t], 0) // 2, 0, 0)),
                pl.BlockSpec((1, hid, emb), lambda t, tb, *_: (jnp.maximum(tb[t], 0) // 2, 0, 0)),
                pl.BlockSpec((1, 1, emb), lambda t, tb, *_: (jnp.maximum(tb[t], 0) // 2, 0, 0)),
            ],
            out_specs=pl.BlockSpec((tt, emb), lambda t, *_: (t, 0)),
            scratch_shapes=[
                pltpu.VMEM((s_slots, emb), bf16),
                pltpu.VMEM((s_slots, emb), bf16),
                pltpu.SemaphoreType.DMA,
            ],
        ),
        out_shape=jax.ShapeDtypeStruct((s_slots, emb), f32),
        compiler_params=pltpu.CompilerParams(
            vmem_limit_bytes=64 * 1024 * 1024),
    )(tb, offt, endt, rend, qs, ks, vs, xs, outwt, outb3, ln2g3, ln2b3,
      w1t, b13, w2t, b23)

    yg = _sc_gather_y(ys, dstg2)

    rt = min(1024, n)
    nrt = n // rt
    out = pl.pallas_call(
        _combine_kernel,
        grid=(nrt,),
        in_specs=[
            pl.BlockSpec((rt, emb), lambda t: (t, 0)),
            pl.BlockSpec((rt, emb), lambda t: (t + nrt, 0)),
            pl.BlockSpec((rt, 2), lambda t: (t, 0)),
        ],
        out_specs=pl.BlockSpec((rt, emb), lambda t: (t, 0)),
        out_shape=jax.ShapeDtypeStruct((n, emb), f32),
    )(yg, yg, probs2)

    return out.reshape(bsz, n, emb)
